# baseline (speedup 1.0000x reference)
import functools

import jax
import jax.numpy as jnp
from jax import lax
from jax.experimental import pallas as pl
from jax.experimental.pallas import tpu as pltpu
from jax.experimental.pallas import tpu_sc as plsc

_NUM_CORES = 2
_NUM_SUBCORES = 16
_NUM_WORKERS = _NUM_CORES * _NUM_SUBCORES
_NBUF = 4
_LEAD = 2


def _lookup(flat_idx, remap, table, bdim, sdim):
    n = flat_idx.shape[0]
    d = table.shape[1]
    per_w = n // _NUM_WORKERS
    rows_w = per_w // sdim
    assert per_w * _NUM_WORKERS == n and rows_w * sdim == per_w
    main_lo, main_hi = _NBUF - _LEAD, rows_w - _LEAD
    assert (main_hi - main_lo) % _NBUF == 0

    mesh = plsc.VectorSubcoreMesh(core_axis_name="c", subcore_axis_name="s")

    @functools.partial(
        pl.kernel,
        out_type=jax.ShapeDtypeStruct((bdim, sdim, d), jnp.float32),
        mesh=mesh,
        compiler_params=pltpu.CompilerParams(use_tc_tiling_on_sc=False),
        scratch_types=[
            pltpu.VMEM((per_w,), jnp.int32),
            pltpu.VMEM((per_w,), jnp.int32),
            pltpu.VMEM((_NBUF, 1, sdim, d), jnp.float32),
            pltpu.SemaphoreType.DMA,
            pltpu.SemaphoreType.DMA((_NBUF,)),
            pltpu.SemaphoreType.DMA((_NBUF,)),
        ],
    )
    def run(idx_hbm, remap_hbm, table_hbm, out_hbm, idx_v, emb_v, rows_v,
            sem_in, sem_g, sem_s):
        wid = lax.axis_index("s") * _NUM_CORES + lax.axis_index("c")
        base = wid * per_w
        base_row = wid * rows_w

        pltpu.sync_copy(idx_hbm.at[pl.ds(base, per_w)], idx_v)
        pltpu.async_copy(remap_hbm.at[idx_v], emb_v, sem_in).wait()

        def gd(g, b):
            return pltpu.make_async_copy(
                table_hbm.at[emb_v.at[pl.ds(g * sdim, sdim)]],
                rows_v.at[b, 0], sem_g.at[b])

        def sd(g, b):
            return pltpu.make_async_copy(
                rows_v.at[b], out_hbm.at[pl.ds(base_row + g, 1)],
                sem_s.at[b])

        for g in range(_LEAD):
            gd(g, g % _NBUF).start()

        def visit(g, b, store_wait, restart):
            gd(g, b).wait()
            sd(g, b).start()
            if restart:
                bn = (g + _LEAD) % _NBUF
                if store_wait:
                    sd(g + _LEAD - _NBUF, bn).wait()
                gd(g + _LEAD, bn).start()
            elif store_wait:
                bn = (g + _LEAD) % _NBUF
                sd(g + _LEAD - _NBUF, bn).wait()

        for g in range(main_lo):
            visit(g, g % _NBUF, store_wait=False, restart=True)

        def body(i, carry):
            g0 = main_lo + i * _NBUF
            for j in range(_NBUF):
                visit(g0 + j, (main_lo + j) % _NBUF, store_wait=True,
                      restart=True)
            return carry

        lax.fori_loop(0, (main_hi - main_lo) // _NBUF, body, 0)

        for g in range(main_hi, rows_w):
            visit(g, g % _NBUF, store_wait=True, restart=False)

        for g in range(main_hi, rows_w):
            sd(g, g % _NBUF).wait()

    return run(flat_idx, remap, table)


def kernel(raw_seqs, remap, table):
    b, s = raw_seqs.shape
    return _lookup(raw_seqs.reshape(b * s), remap, table, b, s)


# speedup vs baseline: 1.2732x; 1.2732x over previous
import functools

import jax
import jax.numpy as jnp
from jax import lax
from jax.experimental import pallas as pl
from jax.experimental.pallas import tpu as pltpu
from jax.experimental.pallas import tpu_sc as plsc

_NUM_CORES = 2
_NUM_SUBCORES = 16
_NUM_WORKERS = _NUM_CORES * _NUM_SUBCORES
_CHUNK = 128
_NBUF = 4
_LEAD = 2


def _lookup(idx3, remap, table_p):
    nw, rows_w, ck = idx3.shape
    t, dp = table_p.shape
    d = dp // 2
    n = nw * rows_w * ck
    assert nw == _NUM_WORKERS and ck == _CHUNK
    per_w = rows_w * ck
    main_lo, main_hi = _NBUF - _LEAD, rows_w - _LEAD
    assert (main_hi - main_lo) % _NBUF == 0

    mesh = plsc.VectorSubcoreMesh(core_axis_name="c", subcore_axis_name="s")

    @functools.partial(
        pl.kernel,
        out_type=jax.ShapeDtypeStruct((n, dp), jnp.float32),
        mesh=mesh,
        scratch_types=[
            pltpu.VMEM((rows_w, ck), jnp.int32),
            pltpu.VMEM((rows_w, ck), jnp.int32),
            pltpu.VMEM((_NBUF, ck, dp), jnp.float32),
            pltpu.SemaphoreType.DMA,
            pltpu.SemaphoreType.DMA((_NBUF,)),
            pltpu.SemaphoreType.DMA((_NBUF,)),
        ],
    )
    def run(idx_hbm, remap_hbm, table_hbm, out_hbm, idx_v, emb_v, rows_v,
            sem_in, sem_g, sem_s):
        wid = lax.axis_index("s") * _NUM_CORES + lax.axis_index("c")
        base = wid * per_w

        pltpu.sync_copy(idx_hbm.at[wid], idx_v)

        def rd(r):
            return pltpu.make_async_copy(
                remap_hbm.at[idx_v.at[r]], emb_v.at[r], sem_in)

        rk = 8
        for r in range(rk):
            rd(r).start()

        def rbody(i, carry):
            rd(i + rk).start()
            rd(i).wait()
            return carry

        lax.fori_loop(0, rows_w - rk, rbody, 0)
        for r in range(rk):
            rd(rows_w - rk + r).wait()

        def gd(g, b):
            return pltpu.make_async_copy(
                table_hbm.at[emb_v.at[g]], rows_v.at[b], sem_g.at[b])

        def sd(g, b):
            return pltpu.make_async_copy(
                rows_v.at[b],
                out_hbm.at[pl.ds(base + g * ck, ck)],
                sem_s.at[b])

        for g in range(_LEAD):
            gd(g, g % _NBUF).start()

        def visit(g, b, store_wait, restart):
            gd(g, b).wait()
            sd(g, b).start()
            if restart:
                bn = (g + _LEAD) % _NBUF
                if store_wait:
                    sd(g + _LEAD - _NBUF, bn).wait()
                gd(g + _LEAD, bn).start()
            elif store_wait:
                bn = (g + _LEAD) % _NBUF
                sd(g + _LEAD - _NBUF, bn).wait()

        for g in range(main_lo):
            visit(g, g % _NBUF, store_wait=False, restart=True)

        def body(i, carry):
            g0 = main_lo + i * _NBUF
            for j in range(_NBUF):
                visit(g0 + j, (main_lo + j) % _NBUF, store_wait=True,
                      restart=True)
            return carry

        lax.fori_loop(0, (main_hi - main_lo) // _NBUF, body, 0)

        for g in range(main_hi, rows_w):
            visit(g, g % _NBUF, store_wait=True, restart=False)

        for g in range(main_hi, rows_w):
            sd(g, g % _NBUF).wait()

    return run(idx3, remap, table_p)


def kernel(raw_seqs, remap, table):
    b, s = raw_seqs.shape
    t, d = table.shape
    n = b * s
    idx3 = raw_seqs.reshape(_NUM_WORKERS, n // (_NUM_WORKERS * _CHUNK), _CHUNK)
    table_p = jnp.pad(table, ((0, 0), (0, d)))
    out = _lookup(idx3, remap, table_p)
    return out[:, :d].reshape(b, s, d)


# speedup vs baseline: 1.2814x; 1.0064x over previous
import functools

import jax
import jax.numpy as jnp
from jax import lax
from jax.experimental import pallas as pl
from jax.experimental.pallas import tpu as pltpu
from jax.experimental.pallas import tpu_sc as plsc

_NUM_CORES = 2
_NUM_SUBCORES = 16
_NUM_WORKERS = _NUM_CORES * _NUM_SUBCORES
_CHUNK = 128
_NBUF = 4
_LEAD = 3


def _lookup(idx3, remap, table_p):
    nw, rows_w, ck = idx3.shape
    t, dp = table_p.shape
    d = dp // 2
    n = nw * rows_w * ck
    assert nw == _NUM_WORKERS and ck == _CHUNK
    per_w = rows_w * ck
    main_lo, main_hi = _NBUF - _LEAD, rows_w - _LEAD
    main_hi -= (main_hi - main_lo) % _NBUF
    assert (main_hi - main_lo) % _NBUF == 0

    mesh = plsc.VectorSubcoreMesh(core_axis_name="c", subcore_axis_name="s")

    @functools.partial(
        pl.kernel,
        out_type=jax.ShapeDtypeStruct((n, dp), jnp.float32),
        mesh=mesh,
        scratch_types=[
            pltpu.VMEM((rows_w, ck), jnp.int32),
            pltpu.VMEM((rows_w, ck), jnp.int32),
            pltpu.VMEM((_NBUF, ck, dp), jnp.float32),
            pltpu.SemaphoreType.DMA,
            pltpu.SemaphoreType.DMA((_NBUF,)),
            pltpu.SemaphoreType.DMA((_NBUF,)),
        ],
    )
    def run(idx_hbm, remap_hbm, table_hbm, out_hbm, idx_v, emb_v, rows_v,
            sem_in, sem_g, sem_s):
        wid = lax.axis_index("s") * _NUM_CORES + lax.axis_index("c")
        base = wid * per_w

        pltpu.sync_copy(idx_hbm.at[wid], idx_v)

        def rd(r):
            return pltpu.make_async_copy(
                remap_hbm.at[idx_v.at[r]], emb_v.at[r], sem_in)

        rk = 16
        for r in range(rk):
            rd(r).start()

        def rbody(i, carry):
            rd(i + rk).start()
            rd(i).wait()
            return carry

        lax.fori_loop(0, rows_w - rk, rbody, 0)
        for r in range(rk):
            rd(rows_w - rk + r).wait()

        def gd(g, b):
            return pltpu.make_async_copy(
                table_hbm.at[emb_v.at[g]], rows_v.at[b], sem_g.at[b])

        def sd(g, b):
            return pltpu.make_async_copy(
                rows_v.at[b],
                out_hbm.at[pl.ds(base + g * ck, ck)],
                sem_s.at[b])

        for g in range(_LEAD):
            gd(g, g % _NBUF).start()

        def visit(g, b, store_wait, restart):
            gd(g, b).wait()
            sd(g, b).start()
            if restart:
                bn = (g + _LEAD) % _NBUF
                if store_wait:
                    sd(g + _LEAD - _NBUF, bn).wait()
                gd(g + _LEAD, bn).start()
            elif store_wait:
                bn = (g + _LEAD) % _NBUF
                sd(g + _LEAD - _NBUF, bn).wait()

        for g in range(main_lo):
            visit(g, g % _NBUF, store_wait=False, restart=True)

        def body(i, carry):
            g0 = main_lo + i * _NBUF
            for j in range(_NBUF):
                visit(g0 + j, (main_lo + j) % _NBUF, store_wait=True,
                      restart=True)
            return carry

        lax.fori_loop(0, (main_hi - main_lo) // _NBUF, body, 0)

        for g in range(main_hi, rows_w):
            visit(g, g % _NBUF, store_wait=True, restart=False)

        for g in range(rows_w - (_NBUF - _LEAD), rows_w):
            sd(g, g % _NBUF).wait()

    return run(idx3, remap, table_p)


def kernel(raw_seqs, remap, table):
    b, s = raw_seqs.shape
    t, d = table.shape
    n = b * s
    idx3 = raw_seqs.reshape(_NUM_WORKERS, n // (_NUM_WORKERS * _CHUNK), _CHUNK)
    table_p = jnp.pad(table, ((0, 0), (0, d)))
    out = _lookup(idx3, remap, table_p)
    return out[:, :d].reshape(b, s, d)


# speedup vs baseline: 1.2921x; 1.0084x over previous
import functools

import jax
import jax.numpy as jnp
from jax import lax
from jax.experimental import pallas as pl
from jax.experimental.pallas import tpu as pltpu
from jax.experimental.pallas import tpu_sc as plsc

_NUM_CORES = 2
_NUM_SUBCORES = 16
_NUM_WORKERS = _NUM_CORES * _NUM_SUBCORES
_CHUNK = 128
_NBUF = 4
_LEAD = 3


def _lookup(idx3, remap, table_p):
    nw, per_w = idx3.shape
    rows_w, ck = per_w // _CHUNK, _CHUNK
    t, dp = table_p.shape
    d = dp // 2
    n = nw * per_w
    assert nw == _NUM_WORKERS
    main_lo, main_hi = _NBUF - _LEAD, rows_w - _LEAD
    main_hi -= (main_hi - main_lo) % _NBUF
    assert (main_hi - main_lo) % _NBUF == 0

    mesh = plsc.VectorSubcoreMesh(core_axis_name="c", subcore_axis_name="s")

    @functools.partial(
        pl.kernel,
        out_type=jax.ShapeDtypeStruct((n, dp), jnp.float32),
        mesh=mesh,
        scratch_types=[
            pltpu.VMEM((per_w,), jnp.int32),
            pltpu.VMEM((per_w,), jnp.int32),
            pltpu.VMEM((_NBUF, ck, dp), jnp.float32),
            pltpu.SemaphoreType.DMA,
            pltpu.SemaphoreType.DMA((_NBUF,)),
            pltpu.SemaphoreType.DMA((_NBUF,)),
        ],
    )
    def run(idx_hbm, remap_hbm, table_hbm, out_hbm, idx_v, emb_v, rows_v,
            sem_in, sem_g, sem_s):
        wid = lax.axis_index("s") * _NUM_CORES + lax.axis_index("c")
        base = wid * per_w

        pltpu.sync_copy(idx_hbm.at[wid], idx_v)
        pltpu.async_copy(remap_hbm.at[idx_v], emb_v, sem_in).wait()

        def gd(g, b):
            return pltpu.make_async_copy(
                table_hbm.at[emb_v.at[pl.ds(g * ck, ck)]], rows_v.at[b], sem_g.at[b])

        def sd(g, b):
            return pltpu.make_async_copy(
                rows_v.at[b],
                out_hbm.at[pl.ds(base + g * ck, ck)],
                sem_s.at[b])

        for g in range(_LEAD):
            gd(g, g % _NBUF).start()

        def visit(g, b, store_wait, restart):
            gd(g, b).wait()
            sd(g, b).start()
            if restart:
                bn = (g + _LEAD) % _NBUF
                if store_wait:
                    sd(g + _LEAD - _NBUF, bn).wait()
                gd(g + _LEAD, bn).start()
            elif store_wait:
                bn = (g + _LEAD) % _NBUF
                sd(g + _LEAD - _NBUF, bn).wait()

        for g in range(main_lo):
            visit(g, g % _NBUF, store_wait=False, restart=True)

        def body(i, carry):
            g0 = main_lo + i * _NBUF
            for j in range(_NBUF):
                visit(g0 + j, (main_lo + j) % _NBUF, store_wait=True,
                      restart=True)
            return carry

        lax.fori_loop(0, (main_hi - main_lo) // _NBUF, body, 0)

        for g in range(main_hi, rows_w):
            visit(g, g % _NBUF, store_wait=True, restart=False)

        for g in range(rows_w - (_NBUF - _LEAD), rows_w):
            sd(g, g % _NBUF).wait()

    return run(idx3, remap, table_p)


def kernel(raw_seqs, remap, table):
    b, s = raw_seqs.shape
    t, d = table.shape
    n = b * s
    idx3 = raw_seqs.reshape(_NUM_WORKERS, n // _NUM_WORKERS)
    table_p = jnp.pad(table, ((0, 0), (0, d)))
    out = _lookup(idx3, remap, table_p)
    return out[:, :d].reshape(b, s, d)
